# Initial kernel scaffold; baseline (speedup 1.0000x reference)
import jax
import jax.numpy as jnp
from jax.experimental import pallas as pl


def kernel(prediction, GT, CL, CE):
    raise NotImplementedError("write your pallas kernel here")


# speedup vs baseline: 33.9281x; 33.9281x over previous
import functools

import jax
import jax.numpy as jnp
from jax import lax
from jax.experimental import pallas as pl
from jax.experimental.pallas import tpu as pltpu
from jax.experimental.pallas import tpu_sc as plsc

B = 2
ROWS = 4096
LANES = 128
P = ROWS * LANES
CHR = 512
NCH = ROWS // CHR

K = 16384
NSLOT = 8
NB = NSLOT * 2 * K
KROWS = K // LANES

SC_NC = 2
SC_NS = 16
NW = SC_NC * SC_NS
N_ELEM = NSLOT * P
SPAN = N_ELEM // NW
SC_CH = 16384
SC_NCHUNK = SPAN // SC_CH
STRIPE = NB // SC_NS

FMAX = 3.4028235e38
NSTAT = 14


def _xyz(ch):
    ri = lax.broadcasted_iota(jnp.int32, (CHR, LANES), 0) + ch * CHR
    li = lax.broadcasted_iota(jnp.int32, (CHR, LANES), 1)
    z = ri // 128
    y = ri - z * 128
    xm = li.astype(jnp.float32) * (1.0 / 127.0)
    ym = y.astype(jnp.float32) * (1.0 / 127.0)
    zm = z.astype(jnp.float32) * (1.0 / 31.0)
    return xm, ym, zm


def _p1_body(sig_ref, seed_ref, gt_ref, cl_ref, ce_ref, out_ref, acc):
    ch = pl.program_id(1)

    @pl.when(ch == 0)
    def _init():
        for j in range(64):
            acc[j] = 0.0

    xm, ym, zm = _xyz(ch)
    gt = gt_ref[0]
    ce_f = (ce_ref[0] != 0).astype(jnp.float32)
    cl_f = (cl_ref[0] != 0).astype(jnp.float32)
    seed = jax.nn.sigmoid(seed_ref[0])
    acc[56] = acc[56] + jnp.sum(seed * seed * (1.0 - cl_f))
    sx = sig_ref[0, 0]
    sy = sig_ref[0, 1]
    sz = sig_ref[0, 2]
    for i in range(4):
        mf = (gt == i + 1).astype(jnp.float32)
        cm = mf * ce_f
        o = i * NSTAT
        acc[o + 0] = acc[o + 0] + jnp.sum(mf)
        acc[o + 1] = acc[o + 1] + jnp.sum(cm)
        acc[o + 2] = acc[o + 2] + jnp.sum(xm * cm)
        acc[o + 3] = acc[o + 3] + jnp.sum(ym * cm)
        acc[o + 4] = acc[o + 4] + jnp.sum(zm * cm)
        acc[o + 5] = acc[o + 5] + jnp.sum(xm * mf)
        acc[o + 6] = acc[o + 6] + jnp.sum(ym * mf)
        acc[o + 7] = acc[o + 7] + jnp.sum(zm * mf)
        acc[o + 8] = acc[o + 8] + jnp.sum(sx * mf)
        acc[o + 9] = acc[o + 9] + jnp.sum(sy * mf)
        acc[o + 10] = acc[o + 10] + jnp.sum(sz * mf)
        acc[o + 11] = acc[o + 11] + jnp.sum(sx * sx * mf)
        acc[o + 12] = acc[o + 12] + jnp.sum(sy * sy * mf)
        acc[o + 13] = acc[o + 13] + jnp.sum(sz * sz * mf)

    @pl.when(ch == NCH - 1)
    def _flush():
        for j in range(64):
            out_ref[0, 0, j] = acc[j]


def _pass1(sig, seedr, gt, cl, ce):
    return pl.pallas_call(
        _p1_body,
        grid=(B, NCH),
        in_specs=[
            pl.BlockSpec((1, 3, CHR, LANES), lambda b, ch: (b, 0, ch, 0)),
            pl.BlockSpec((1, CHR, LANES), lambda b, ch: (b, ch, 0)),
            pl.BlockSpec((1, CHR, LANES), lambda b, ch: (b, ch, 0)),
            pl.BlockSpec((1, CHR, LANES), lambda b, ch: (b, ch, 0)),
            pl.BlockSpec((1, CHR, LANES), lambda b, ch: (b, ch, 0)),
        ],
        out_specs=pl.BlockSpec((1, 1, 64), lambda b, ch: (b, 0, 0),
                               memory_space=pltpu.SMEM),
        out_shape=jax.ShapeDtypeStruct((B, 1, 64), jnp.float32),
        scratch_shapes=[pltpu.SMEM((64,), jnp.float32)],
    )(sig, seedr, gt, cl, ce)


def _p2_body(emb_ref, seed_ref, gt_ref, stats_ref, bin_ref, sfg_ref, acc):
    b = pl.program_id(0)
    ch = pl.program_id(1)

    @pl.when(ch == 0)
    def _init():
        for j in range(8):
            acc[j] = 0.0

    xm, ym, zm = _xyz(ch)
    ex = jnp.tanh(emb_ref[0, 0]) + xm
    ey = jnp.tanh(emb_ref[0, 1]) + ym
    ez = jnp.tanh(emb_ref[0, 2]) + zm
    seed = jax.nn.sigmoid(seed_ref[0])
    gt = gt_ref[0]
    kf = jnp.float32(K)
    for i in range(4):
        o = i * NSTAT
        cnt = stats_ref[0, 0, o + 0]
        ccnt = stats_ref[0, 0, o + 1]
        safe = jnp.maximum(cnt, 1.0)
        one_c = ccnt == 1.0
        cx = jnp.where(one_c, stats_ref[0, 0, o + 2], stats_ref[0, 0, o + 5] / safe)
        cy = jnp.where(one_c, stats_ref[0, 0, o + 3], stats_ref[0, 0, o + 6] / safe)
        cz = jnp.where(one_c, stats_ref[0, 0, o + 4], stats_ref[0, 0, o + 7] / safe)
        s4x = jnp.minimum(jnp.exp(10.0 * stats_ref[0, 0, o + 8] / safe), FMAX)
        s4y = jnp.minimum(jnp.exp(10.0 * stats_ref[0, 0, o + 9] / safe), FMAX)
        s4z = jnp.minimum(jnp.exp(10.0 * stats_ref[0, 0, o + 10] / safe), FMAX)
        q = ((ex - cx) * (ex - cx) * s4x + (ey - cy) * (ey - cy) * s4y
             + (ez - cz) * (ez - cz) * s4z)
        d = jnp.exp(-q)
        mi = gt == i + 1
        mf = mi.astype(jnp.float32)
        dv = seed - d
        acc[i] = acc[i] + jnp.sum(dv * dv * mf)
        binf = jnp.where(mi, kf - kf * d, kf * d)
        binn = jnp.clip(jnp.floor(binf).astype(jnp.int32), 0, K - 1)
        slotbase = ((b * 4 + i) * 2) * K
        idx = binn + jnp.where(mi, slotbase + K, slotbase)
        bin_ref[0, i] = idx

    @pl.when(ch == NCH - 1)
    def _flush():
        for j in range(8):
            sfg_ref[0, 0, j] = acc[j]


def _pass2(emb, seedr, gt, stats):
    return pl.pallas_call(
        _p2_body,
        grid=(B, NCH),
        in_specs=[
            pl.BlockSpec((1, 3, CHR, LANES), lambda b, ch: (b, 0, ch, 0)),
            pl.BlockSpec((1, CHR, LANES), lambda b, ch: (b, ch, 0)),
            pl.BlockSpec((1, CHR, LANES), lambda b, ch: (b, ch, 0)),
            pl.BlockSpec((1, 1, 64), lambda b, ch: (b, 0, 0),
                         memory_space=pltpu.SMEM),
        ],
        out_specs=[
            pl.BlockSpec((1, 4, CHR, LANES), lambda b, ch: (b, 0, ch, 0)),
            pl.BlockSpec((1, 1, 8), lambda b, ch: (b, 0, 0),
                         memory_space=pltpu.SMEM),
        ],
        out_shape=[
            jax.ShapeDtypeStruct((B, 4, ROWS, LANES), jnp.int32),
            jax.ShapeDtypeStruct((B, 1, 8), jnp.float32),
        ],
        scratch_shapes=[pltpu.SMEM((8,), jnp.float32)],
    )(emb, seedr, gt, stats)


def _sc_hist_body(idx_hbm, zeros_hbm, ones_hbm, out_hbm,
                  hist_sh, idxv, onesv, stagev):
    c = lax.axis_index("c")
    s = lax.axis_index("s")
    wid = s * SC_NC + c
    pltpu.sync_copy(zeros_hbm, stagev)
    pltpu.sync_copy(stagev, hist_sh.at[pl.ds(s * STRIPE, STRIPE)])
    pltpu.sync_copy(ones_hbm, onesv)
    plsc.subcore_barrier()
    base = wid * SPAN
    for j in range(SC_NCHUNK):
        pltpu.sync_copy(idx_hbm.at[pl.ds(base + j * SC_CH, SC_CH)], idxv)
        pltpu.sync_copy(onesv, hist_sh.at[idxv], add=True)
    plsc.subcore_barrier()
    pltpu.sync_copy(hist_sh.at[pl.ds(s * STRIPE, STRIPE)], stagev)
    pltpu.sync_copy(stagev, out_hbm.at[c, pl.ds(s * STRIPE, STRIPE)])


def _sc_histogram(idx_flat, zeros, ones):
    mesh = plsc.VectorSubcoreMesh(core_axis_name="c", subcore_axis_name="s")
    kern = functools.partial(
        pl.kernel,
        mesh=mesh,
        out_type=jax.ShapeDtypeStruct((SC_NC, NB), jnp.float32),
        scratch_types=[
            pltpu.VMEM_SHARED((NB,), jnp.float32),
            pltpu.VMEM((SC_CH,), jnp.int32),
            pltpu.VMEM((SC_CH,), jnp.float32),
            pltpu.VMEM((STRIPE,), jnp.float32),
        ],
    )(_sc_hist_body)
    return kern(idx_flat, zeros, ones)


def _fin_body(hist_ref, stats_ref, sfg_ref, out_ref):
    hist = hist_ref[0] + hist_ref[1]
    ki = lax.broadcasted_iota(jnp.int32, (LANES, LANES), 0)
    ci = lax.broadcasted_iota(jnp.int32, (LANES, LANES), 1)
    t_suf = (ki >= ci).astype(jnp.float32)
    a_suf = (ci > ki).astype(jnp.float32)
    wbin = jnp.float32(2.0 / K)
    total = jnp.float32(0.0)
    for b in range(B):
        inst = jnp.float32(0.0)
        var = jnp.float32(0.0)
        seedl = stats_ref[b, 0, 56]
        obj = jnp.float32(0.0)
        for i in range(4):
            sl = b * 4 + i
            r0 = sl * 2 * KROWS
            n0 = hist[r0:r0 + KROWS, :]
            n1 = hist[r0 + KROWS:r0 + 2 * KROWS, :]
            tot = n0 + n1
            wc = jnp.dot(tot, t_suf, preferred_element_type=jnp.float32)
            wg = jnp.dot(n1, t_suf, preferred_element_type=jnp.float32)
            rc = jnp.dot(a_suf, wc[:, 0:1], preferred_element_type=jnp.float32)
            rg = jnp.dot(a_suf, wg[:, 0:1], preferred_element_type=jnp.float32)
            sc_ = wc + rc
            sg = wg + rg
            o = i * NSTAT
            cnt = stats_ref[b, 0, o + 0]
            present = (cnt > 0.0).astype(jnp.float32)
            safe = jnp.maximum(cnt, 1.0)
            g_tot = cnt
            h = (g_tot - sg) / jnp.maximum(g_tot + sc_ - sg, 1.0)
            h_k = g_tot / jnp.maximum(g_tot, 1.0)
            hsum = jnp.sum(h) + h_k
            s_lov = 2.0 - wbin * (hsum - 0.5 * h_k)
            inst = inst + present * s_lov
            vs = jnp.float32(0.0)
            for k in range(3):
                ssum = stats_ref[b, 0, o + 8 + k]
                s2sum = stats_ref[b, 0, o + 11 + k]
                sm = ssum / safe
                vs = vs + (s2sum - 2.0 * sm * ssum + sm * sm * cnt)
            var = var + present * vs / (3.0 * safe)
            seedl = seedl + present * 10.0 * sfg_ref[b, 0, i]
            obj = obj + present
        so = jnp.maximum(obj, 1.0)
        total = total + inst / so + 10.0 * var / so + seedl / jnp.float32(P)
    out_ref[0] = total * jnp.float32(1.0 / B)


def _finalize(hist, stats, sfg):
    return pl.pallas_call(
        _fin_body,
        grid=(1,),
        in_specs=[
            pl.BlockSpec((SC_NC, NB // LANES, LANES), lambda _: (0, 0, 0)),
            pl.BlockSpec((B, 1, 64), lambda _: (0, 0, 0),
                         memory_space=pltpu.SMEM),
            pl.BlockSpec((B, 1, 8), lambda _: (0, 0, 0),
                         memory_space=pltpu.SMEM),
        ],
        out_specs=pl.BlockSpec((1,), lambda _: (0,), memory_space=pltpu.SMEM),
        out_shape=jax.ShapeDtypeStruct((1,), jnp.float32),
    )(hist, stats, sfg)


def kernel(prediction, GT, CL, CE):
    emb = prediction[:, 0:3].reshape(B, 3, ROWS, LANES)
    sig = prediction[:, 3:6].reshape(B, 3, ROWS, LANES)
    seedr = prediction[:, 6].reshape(B, ROWS, LANES)
    gt = GT.reshape(B, ROWS, LANES).astype(jnp.int32)
    cl = CL.reshape(B, ROWS, LANES).astype(jnp.int32)
    ce = CE.reshape(B, ROWS, LANES).astype(jnp.int32)

    stats = _pass1(sig, seedr, gt, cl, ce)
    binidx, sfg = _pass2(emb, seedr, gt, stats)
    zeros = jnp.zeros((STRIPE,), jnp.float32)
    ones = jnp.ones((SC_CH,), jnp.float32)
    hist = _sc_histogram(binidx.reshape(N_ELEM), zeros, ones)
    out = _finalize(hist.reshape(SC_NC, NB // LANES, LANES), stats, sfg)
    return out[0]
